# baseline (speedup 1.0000x reference)
import functools

import jax
import jax.numpy as jnp
from jax import lax
from jax.experimental import pallas as pl
from jax.experimental.pallas import tpu as pltpu
from jax.experimental.pallas import tpu_sc as plsc

_NUM_CLASSES = 100
_BATCH = 4096
_LAMBDA = 3.0
_NUM_WORKERS = 32
_B_PER_W = _BATCH // _NUM_WORKERS


def _sc_gather(history, idx):
    mesh = plsc.VectorSubcoreMesh(core_axis_name="c", subcore_axis_name="s")

    @functools.partial(
        pl.kernel,
        out_type=jax.ShapeDtypeStruct((_BATCH, _NUM_CLASSES), jnp.float32),
        mesh=mesh,
        scratch_types=[
            pltpu.VMEM((_B_PER_W,), jnp.int32),
            pltpu.VMEM((_B_PER_W, _NUM_CLASSES), jnp.float32),
            pltpu.SemaphoreType.DMA,
        ],
    )
    def gather_kernel(hist_hbm, idx_hbm, out_hbm, idx_v, rows_v, sem):
        wid = lax.axis_index("s") * 2 + lax.axis_index("c")
        base = wid * _B_PER_W
        pltpu.sync_copy(idx_hbm.at[pl.ds(base, _B_PER_W)], idx_v)

        if True:
            pltpu.sync_copy(hist_hbm.at[pl.ds(0, _B_PER_W)], rows_v)
            pltpu.sync_copy(rows_v, out_hbm.at[pl.ds(base, _B_PER_W)])
            return

        def issue(k, _):
            v = idx_v[pl.ds(k * 16, 16)]
            for j in range(16):
                pltpu.async_copy(hist_hbm.at[v[j]], rows_v.at[k * 16 + j], sem)
            return 0

        lax.fori_loop(0, _B_PER_W // 16, issue, 0)
        pltpu.make_async_copy(
            hist_hbm.at[pl.ds(0, _B_PER_W)], rows_v, sem
        ).wait()
        pltpu.sync_copy(rows_v, out_hbm.at[pl.ds(base, _B_PER_W)])

    return gather_kernel(history, idx)


def _tc_loss_body(out_ref, tgt_ref, hist_ref, loss_ref):
    x = out_ref[...]
    m = jnp.max(x, axis=1, keepdims=True)
    xm = x - m
    e = jnp.exp(xm)
    s = jnp.sum(e, axis=1, keepdims=True)
    y = jnp.clip(e / s, 0.0001, 1.0 - 0.0001)
    log_sm = xm - jnp.log(s)
    ce = jnp.sum(-tgt_ref[...] * log_sm)
    dot = jnp.sum(hist_ref[...] * y, axis=1, keepdims=True)
    reg = jnp.sum(jnp.log(1.0 - dot))
    loss_ref[0, 0] = (ce + _LAMBDA * reg) / _BATCH


def _tc_loss(output, target, hist_g):
    return pl.pallas_call(
        _tc_loss_body,
        out_shape=jax.ShapeDtypeStruct((1, 1), jnp.float32),
        in_specs=[
            pl.BlockSpec(memory_space=pltpu.VMEM),
            pl.BlockSpec(memory_space=pltpu.VMEM),
            pl.BlockSpec(memory_space=pltpu.VMEM),
        ],
        out_specs=pl.BlockSpec(memory_space=pltpu.SMEM),
    )(output, target, hist_g)


def kernel(index, output, target, history):
    idx = index.astype(jnp.int32)
    hist_g = _sc_gather(history, idx)
    return hist_g[0, 0]


# speedup vs baseline: 3.6444x; 3.6444x over previous
import functools

import jax
import jax.numpy as jnp
from jax import lax
from jax.experimental import pallas as pl
from jax.experimental.pallas import tpu as pltpu
from jax.experimental.pallas import tpu_sc as plsc

_NUM_CLASSES = 100
_BATCH = 4096
_LAMBDA = 3.0
_NUM_WORKERS = 32
_B_PER_W = _BATCH // _NUM_WORKERS
_NSLAB = 8


def _sc_gather(hist_t, idx):
    mesh = plsc.VectorSubcoreMesh(core_axis_name="c", subcore_axis_name="s")

    @functools.partial(
        pl.kernel,
        out_type=jax.ShapeDtypeStruct((_BATCH, _NUM_CLASSES), jnp.float32),
        mesh=mesh,
        scratch_types=[
            pltpu.VMEM((_B_PER_W,), jnp.int32),
            [pltpu.VMEM((_NUM_CLASSES, 128), jnp.float32)] * _NSLAB,
            pltpu.VMEM((_B_PER_W, _NUM_CLASSES), jnp.float32),
            pltpu.SemaphoreType.DMA,
        ],
        compiler_params=pltpu.CompilerParams(needs_layout_passes=False),
    )
    def gather_kernel(hist_hbm, idx_hbm, out_hbm, idx_v, slabs_v, compact_v, sem):
        wid = lax.axis_index("s") * 2 + lax.axis_index("c")
        base = wid * _B_PER_W
        pltpu.sync_copy(idx_hbm.at[pl.ds(base, _B_PER_W)], idx_v)

        iota16 = lax.iota(jnp.int32, 16)
        chunk_starts = [0, 16, 32, 48, 64, 80, 84]

        def body(k, _):
            v = idx_v[pl.ds(k * 16, 16)]
            for h in (0, 1):
                ts = []
                for jj in range(_NSLAB):
                    r = v[8 * h + jj]
                    t = lax.bitwise_and(r, 127)
                    off = pl.multiple_of(lax.sub(r, t), 128)
                    ts.append(t)
                    pltpu.async_copy(
                        hist_hbm.at[:, pl.ds(off, 128)], slabs_v[jj], sem
                    )
                for jj in range(_NSLAB):
                    pltpu.make_async_copy(
                        hist_hbm.at[:, pl.ds(0, 128)], slabs_v[jj], sem
                    ).wait()
                for jj in range(_NSLAB):
                    row = k * 16 + 8 * h + jj
                    tvec = iota16 * 0 + ts[jj]
                    for c0 in chunk_starts:
                        vals = plsc.load_gather(
                            slabs_v[jj], [c0 + iota16, tvec]
                        )
                        compact_v[row, pl.ds(c0, 16)] = vals
            return 0

        lax.fori_loop(0, _B_PER_W // 16, body, 0)
        pltpu.sync_copy(compact_v, out_hbm.at[pl.ds(base, _B_PER_W)])

    return gather_kernel(hist_t, idx)


def _tc_loss_body(out_ref, tgt_ref, hist_ref, loss_ref):
    x = out_ref[...]
    m = jnp.max(x, axis=1, keepdims=True)
    xm = x - m
    e = jnp.exp(xm)
    s = jnp.sum(e, axis=1, keepdims=True)
    y = jnp.clip(e / s, 0.0001, 1.0 - 0.0001)
    log_sm = xm - jnp.log(s)
    ce = jnp.sum(-tgt_ref[...] * log_sm)
    dot = jnp.sum(hist_ref[...] * y, axis=1, keepdims=True)
    reg = jnp.sum(jnp.log(1.0 - dot))
    loss_ref[0, 0] = (ce + _LAMBDA * reg) / _BATCH


def _tc_loss(output, target, hist_g):
    return pl.pallas_call(
        _tc_loss_body,
        out_shape=jax.ShapeDtypeStruct((1, 1), jnp.float32),
        in_specs=[
            pl.BlockSpec(memory_space=pltpu.VMEM),
            pl.BlockSpec(memory_space=pltpu.VMEM),
            pl.BlockSpec(memory_space=pltpu.VMEM),
        ],
        out_specs=pl.BlockSpec(memory_space=pltpu.SMEM),
    )(output, target, hist_g)


def kernel(index, output, target, history):
    idx = index.astype(jnp.int32)
    hist_g = _sc_gather(history.T, idx)
    loss = _tc_loss(output, target, hist_g)
    return loss[0, 0]
